# baseline (speedup 1.0000x reference)
import functools

import jax
import jax.numpy as jnp
from jax import lax
from jax.experimental import pallas as pl
from jax.experimental.pallas import tpu as pltpu
from jax.experimental.pallas import tpu_sc as plsc

N = 10000
DIN = 128
FC = 256
G = 64

NC = 2
NS = 16
NW = NC * NS
NPAD = 10240
RPW = NPAD // NW
CHUNK = 80
NCHUNK = RPW // CHUNK

_mesh = plsc.VectorSubcoreMesh(core_axis_name="c", subcore_axis_name="s")


@functools.partial(
    pl.kernel,
    out_type=jax.ShapeDtypeStruct((NC, G, DIN), jnp.float32),
    mesh=_mesh,
    scratch_types=[
        pltpu.VMEM((NCHUNK, CHUNK), jnp.int32),
        pltpu.VMEM((RPW, DIN), jnp.float32),
        pltpu.VMEM_SHARED((G, DIN), jnp.float32),
    ],
)
def _sc_segsum(x_hbm, batch_hbm, zero_hbm, sum_out, idx_v, rows_v, acc_sh):
    cid = lax.axis_index("c")
    sid = lax.axis_index("s")
    wid = cid * NS + sid
    base = wid * RPW

    @pl.when(sid == 0)
    def _():
        pltpu.sync_copy(zero_hbm, acc_sh)

    plsc.subcore_barrier()

    pltpu.sync_copy(x_hbm.at[pl.ds(base, RPW)], rows_v)
    pltpu.sync_copy(batch_hbm.at[wid], idx_v)

    for j in range(NCHUNK):
        pltpu.sync_copy(rows_v.at[pl.ds(j * CHUNK, CHUNK)],
                        acc_sh.at[idx_v.at[j]], add=True)

    plsc.subcore_barrier()

    @pl.when(sid == 0)
    def _():
        pltpu.sync_copy(acc_sh, sum_out.at[cid])


def _finish_kernel(sum_ref, batch_ref, w_ref, b_ref, out_ref):
    s = sum_ref[0] + sum_ref[1]
    ids = jax.lax.broadcasted_iota(jnp.int32, (1, G), 1)
    seg = (batch_ref[...] == ids).astype(jnp.float32)
    cnt = jnp.sum(seg, axis=0)[:, None]
    pooled = s / jnp.maximum(cnt, 1.0)
    out_ref[...] = jnp.dot(pooled, w_ref[...],
                           preferred_element_type=jnp.float32) + b_ref[...]


def kernel(x, edge_index, edge_attr, batch, W_atom, b_atom, W_edge, b_edge):
    del edge_index, edge_attr, W_edge, b_edge
    pad = NPAD - N
    x_pad = jnp.concatenate([x, jnp.zeros((pad, DIN), jnp.float32)], axis=0)
    batch3d = jnp.concatenate(
        [batch, jnp.zeros((pad,), jnp.int32)], axis=0).reshape(NW, NCHUNK, CHUNK)
    zero = jnp.zeros((G, DIN), jnp.float32)
    sums = _sc_segsum(x_pad, batch3d, zero)
    return pl.pallas_call(
        _finish_kernel,
        out_shape=jax.ShapeDtypeStruct((G, FC), jnp.float32),
    )(sums, batch.reshape(N, 1), W_atom, b_atom.reshape(1, FC))


# speedup vs baseline: 1.1384x; 1.1384x over previous
import functools

import jax
import jax.numpy as jnp
from jax import lax
from jax.experimental import pallas as pl
from jax.experimental.pallas import tpu as pltpu
from jax.experimental.pallas import tpu_sc as plsc

N = 10000
DIN = 128
FC = 256
G = 64

NC = 2
NS = 16
NW = NC * NS
RPW = 320
CHUNK = 80
NCHUNK = RPW // CHUNK
NROWCHUNKS = N // CHUNK

_mesh = plsc.VectorSubcoreMesh(core_axis_name="c", subcore_axis_name="s")


@functools.partial(
    pl.kernel,
    out_type=jax.ShapeDtypeStruct((NC, G, DIN), jnp.float32),
    mesh=_mesh,
    scratch_types=[
        pltpu.VMEM((NCHUNK, CHUNK), jnp.int32),
        pltpu.VMEM((RPW, DIN), jnp.float32),
        pltpu.VMEM_SHARED((G, DIN), jnp.float32),
        pltpu.SemaphoreType.DMA,
        pltpu.SemaphoreType.DMA,
    ],
)
def _sc_segsum(x_hbm, batch_hbm, zero_hbm, sum_out,
               idx_v, rows_v, acc_sh, sem_x, sem_i):
    cid = lax.axis_index("c")
    sid = lax.axis_index("s")
    wid = cid * NS + sid
    base = wid * RPW

    @pl.when(sid == 0)
    def _():
        pltpu.sync_copy(zero_hbm, acc_sh)

    idx_cp = pltpu.async_copy(batch_hbm.at[pl.ds(wid * NCHUNK, NCHUNK)],
                              idx_v, sem_i)

    @pl.when(wid < NW - 1)
    def _():
        pltpu.async_copy(x_hbm.at[pl.ds(base, RPW)], rows_v, sem_x).wait()

    @pl.when(wid == NW - 1)
    def _():
        pltpu.async_copy(x_hbm.at[pl.ds(base, CHUNK)],
                         rows_v.at[pl.ds(0, CHUNK)], sem_x).wait()

    idx_cp.wait()
    plsc.subcore_barrier()

    for j in range(NCHUNK):
        @pl.when(wid * NCHUNK + j < NROWCHUNKS)
        def _():
            pltpu.sync_copy(rows_v.at[pl.ds(j * CHUNK, CHUNK)],
                            acc_sh.at[idx_v.at[j]], add=True)

    plsc.subcore_barrier()

    @pl.when(sid == 0)
    def _():
        pltpu.sync_copy(acc_sh, sum_out.at[cid])


def _finish_kernel(sum_ref, batch_ref, w_ref, b_ref, out_ref):
    s = sum_ref[0] + sum_ref[1]
    ids = jax.lax.broadcasted_iota(jnp.int32, (1, G), 1)
    seg = (batch_ref[...] == ids).astype(jnp.float32)
    cnt = jnp.sum(seg, axis=0)[:, None]
    pooled = s / jnp.maximum(cnt, 1.0)
    out_ref[...] = jnp.dot(pooled, w_ref[...],
                           preferred_element_type=jnp.float32) + b_ref[...]


def kernel(x, edge_index, edge_attr, batch, W_atom, b_atom, W_edge, b_edge):
    del edge_index, edge_attr, W_edge, b_edge
    pad = NW * RPW - N
    batch2d = jnp.concatenate(
        [batch, jnp.zeros((pad,), jnp.int32)], axis=0).reshape(NW * NCHUNK, CHUNK)
    zero = jnp.zeros((G, DIN), jnp.float32)
    sums = _sc_segsum(x, batch2d, zero)
    return pl.pallas_call(
        _finish_kernel,
        out_shape=jax.ShapeDtypeStruct((G, FC), jnp.float32),
    )(sums, batch.reshape(N, 1), W_atom, b_atom.reshape(1, FC))


# speedup vs baseline: 2.0366x; 1.7890x over previous
import jax
import jax.numpy as jnp
from jax.experimental import pallas as pl
from jax.experimental.pallas import tpu as pltpu

N = 10000
DIN = 128
FC = 256
G = 64

NBLK = 10
BLK = N // NBLK


def _pool_kernel(x_ref, batch_ref, w_ref, b_ref, out_ref, acc_ref):
    step = pl.program_id(0)
    b2 = batch_ref[...]
    ids = jax.lax.broadcasted_iota(jnp.int32, (1, G), 1)
    seg = (b2 == ids).astype(jnp.float32)
    sums = jax.lax.dot_general(seg, x_ref[...], (((0,), (0,)), ((), ())),
                               preferred_element_type=jnp.float32)
    cnt = jnp.sum(seg, axis=0)[:, None]
    part = jnp.concatenate([sums, cnt], axis=1)

    @pl.when(step == 0)
    def _():
        acc_ref[...] = part

    @pl.when(step > 0)
    def _():
        acc_ref[...] += part

    @pl.when(step == NBLK - 1)
    def _():
        acc = acc_ref[...]
        pooled = acc[:, :DIN] / jnp.maximum(acc[:, DIN:], 1.0)
        out_ref[...] = jnp.dot(pooled, w_ref[...],
                               preferred_element_type=jnp.float32) + b_ref[...]


def kernel(x, edge_index, edge_attr, batch, W_atom, b_atom, W_edge, b_edge):
    del edge_index, edge_attr, W_edge, b_edge
    return pl.pallas_call(
        _pool_kernel,
        grid=(NBLK,),
        in_specs=[
            pl.BlockSpec((BLK, DIN), lambda i: (i, 0)),
            pl.BlockSpec((BLK, 1), lambda i: (i, 0)),
            pl.BlockSpec((DIN, FC), lambda i: (0, 0)),
            pl.BlockSpec((1, FC), lambda i: (0, 0)),
        ],
        out_specs=pl.BlockSpec((G, FC), lambda i: (0, 0)),
        out_shape=jax.ShapeDtypeStruct((G, FC), jnp.float32),
        scratch_shapes=[pltpu.VMEM((G, DIN + 1), jnp.float32)],
    )(x, batch.reshape(N, 1), W_atom, b_atom.reshape(1, FC))


# speedup vs baseline: 2.3471x; 1.1525x over previous
import jax
import jax.numpy as jnp
from jax.experimental import pallas as pl
from jax.experimental.pallas import tpu as pltpu

N = 10000
DIN = 128
FC = 256
G = 64

NBLK = 5
BLK = N // NBLK


def _pool_kernel(x_ref, batch_ref, w_ref, b_ref, out_ref, acc_ref):
    step = pl.program_id(0)
    b2 = batch_ref[...]
    ids = jax.lax.broadcasted_iota(jnp.int32, (1, G), 1)
    seg = (b2 == ids).astype(jnp.float32)
    sums = jax.lax.dot_general(seg, x_ref[...], (((0,), (0,)), ((), ())),
                               preferred_element_type=jnp.float32)
    cnt = jnp.sum(seg, axis=0)[:, None]
    part = jnp.concatenate([sums, cnt], axis=1)

    @pl.when(step == 0)
    def _():
        acc_ref[...] = part

    @pl.when(step > 0)
    def _():
        acc_ref[...] += part

    @pl.when(step == NBLK - 1)
    def _():
        acc = acc_ref[...]
        pooled = acc[:, :DIN] / jnp.maximum(acc[:, DIN:], 1.0)
        out_ref[...] = jnp.dot(pooled, w_ref[...],
                               preferred_element_type=jnp.float32) + b_ref[...]


def kernel(x, edge_index, edge_attr, batch, W_atom, b_atom, W_edge, b_edge):
    del edge_index, edge_attr, W_edge, b_edge
    return pl.pallas_call(
        _pool_kernel,
        grid=(NBLK,),
        in_specs=[
            pl.BlockSpec((BLK, DIN), lambda i: (i, 0)),
            pl.BlockSpec((BLK, 1), lambda i: (i, 0)),
            pl.BlockSpec((DIN, FC), lambda i: (0, 0)),
            pl.BlockSpec((1, FC), lambda i: (0, 0)),
        ],
        out_specs=pl.BlockSpec((G, FC), lambda i: (0, 0)),
        out_shape=jax.ShapeDtypeStruct((G, FC), jnp.float32),
        scratch_shapes=[pltpu.VMEM((G, DIN + 1), jnp.float32)],
    )(x, batch.reshape(N, 1), W_atom, b_atom.reshape(1, FC))
